# baseline (speedup 1.0000x reference)
import jax
import jax.numpy as jnp
from jax.experimental import pallas as pl
from jax.experimental.pallas import tpu as pltpu

N = 20000
DET = 100
SCORE_THRESH = 0.05
NMS_THRESH = 0.5
NEG = -1e9
BIG = 2**31 - 1

ROWS = 160
LANES = 128


def _nms_body(x1_ref, y1_ref, x2_ref, y2_ref, s_ref, out_ref, sc_ref):
    lane = jax.lax.broadcasted_iota(jnp.int32, (1, LANES), 1)
    rowi = jax.lax.broadcasted_iota(jnp.int32, (ROWS, 1), 0)

    sc0 = jnp.where(s_ref[...] > SCORE_THRESH, s_ref[...], NEG)
    sc_ref[...] = sc0
    rowmax0 = jnp.max(sc0, axis=1, keepdims=True)
    m0 = jnp.max(rowmax0)
    r0 = jnp.min(jnp.where(rowmax0 == m0, rowi, BIG))

    def step(i, carry):
        m, r = carry
        scrow = sc_ref[pl.ds(r, 1), :]
        c = jnp.min(jnp.where(scrow == m, lane, BIG))
        lmask = lane == c
        lmf = lmask.astype(jnp.float32)
        bx1 = jnp.sum(x1_ref[pl.ds(r, 1), :] * lmf)
        by1 = jnp.sum(y1_ref[pl.ds(r, 1), :] * lmf)
        bx2 = jnp.sum(x2_ref[pl.ds(r, 1), :] * lmf)
        by2 = jnp.sum(y2_ref[pl.ds(r, 1), :] * lmf)

        valid = (m > SCORE_THRESH).astype(jnp.float32)
        out_ref[pl.ds(i, 1), :] = jnp.where(
            lane == 0, bx1,
            jnp.where(lane == 1, by1,
            jnp.where(lane == 2, bx2,
            jnp.where(lane == 3, by2, m)))) * valid

        sc_ref[pl.ds(r, 1), :] = jnp.where(lmask, NEG, scrow)

        sc = sc_ref[...]
        x1 = x1_ref[...]
        y1 = y1_ref[...]
        x2 = x2_ref[...]
        y2 = y2_ref[...]
        inter = (jnp.maximum(jnp.minimum(bx2, x2) - jnp.maximum(bx1, x1), 0.0)
                 * jnp.maximum(jnp.minimum(by2, y2) - jnp.maximum(by1, y1), 0.0))
        barea = (bx2 - bx1) * (by2 - by1)
        area = (x2 - x1) * (y2 - y1)
        iou = inter / (barea + area - inter + 1e-9)
        sc_new = jnp.where(iou > NMS_THRESH, NEG, sc)
        sc_ref[...] = sc_new

        rowmax = jnp.max(sc_new, axis=1, keepdims=True)
        m2 = jnp.max(rowmax)
        r2 = jnp.min(jnp.where(rowmax == m2, rowi, BIG))
        return (m2, r2)

    jax.lax.fori_loop(0, DET, step, (m0, r0))


@jax.jit
def kernel(boxes, scores):
    pad = ROWS * LANES - N
    x1 = jnp.pad(boxes[:, 0], (0, pad)).reshape(ROWS, LANES)
    y1 = jnp.pad(boxes[:, 1], (0, pad)).reshape(ROWS, LANES)
    x2 = jnp.pad(boxes[:, 2], (0, pad)).reshape(ROWS, LANES)
    y2 = jnp.pad(boxes[:, 3], (0, pad)).reshape(ROWS, LANES)
    s = jnp.pad(scores, (0, pad)).reshape(ROWS, LANES)

    out = pl.pallas_call(
        _nms_body,
        out_shape=jax.ShapeDtypeStruct((DET, LANES), jnp.float32),
        scratch_shapes=[pltpu.VMEM((ROWS, LANES), jnp.float32)],
    )(x1, y1, x2, y2, s)
    return out[:, :5]


# speedup vs baseline: 1.2017x; 1.2017x over previous
import jax
import jax.numpy as jnp
from jax.experimental import pallas as pl
from jax.experimental.pallas import tpu as pltpu

N = 20000
DET = 100
SCORE_THRESH = 0.05
NMS_THRESH = 0.5
NEG = -1e9
BIG = 2**31 - 1

ROWS = 160
LANES = 128


def _nms_body(x1_ref, y1_ref, x2_ref, y2_ref, s_ref, out_ref, sc_ref):
    lane = jax.lax.broadcasted_iota(jnp.int32, (1, LANES), 1)
    flat_iota = (
        jax.lax.broadcasted_iota(jnp.int32, (ROWS, LANES), 0) * LANES
        + jax.lax.broadcasted_iota(jnp.int32, (ROWS, LANES), 1)
    )

    sc_ref[...] = jnp.where(s_ref[...] > SCORE_THRESH, s_ref[...], NEG)

    def step(i, _):
        sc = sc_ref[...]
        m = jnp.max(jnp.max(sc, axis=0, keepdims=True), axis=1, keepdims=True)
        idx = jnp.min(
            jnp.min(jnp.where(sc == m, flat_iota, BIG), axis=0, keepdims=True),
            axis=1, keepdims=True)
        onehot = flat_iota == idx
        def pick(ref):
            v = jnp.where(onehot, ref[...], -1e30)
            return jnp.max(jnp.max(v, axis=0, keepdims=True), axis=1,
                           keepdims=True)
        bx1 = pick(x1_ref)
        by1 = pick(y1_ref)
        bx2 = pick(x2_ref)
        by2 = pick(y2_ref)

        valid = (m > SCORE_THRESH).astype(jnp.float32)
        out_ref[pl.ds(i, 1), :] = jnp.where(
            lane == 0, bx1,
            jnp.where(lane == 1, by1,
            jnp.where(lane == 2, bx2,
            jnp.where(lane == 3, by2, m)))) * valid

        x1 = x1_ref[...]
        y1 = y1_ref[...]
        x2 = x2_ref[...]
        y2 = y2_ref[...]
        inter = (jnp.maximum(jnp.minimum(bx2, x2) - jnp.maximum(bx1, x1), 0.0)
                 * jnp.maximum(jnp.minimum(by2, y2) - jnp.maximum(by1, y1), 0.0))
        barea = (bx2 - bx1) * (by2 - by1)
        area = (x2 - x1) * (y2 - y1)
        iou = inter / (barea + area - inter + 1e-9)
        sc_ref[...] = jnp.where((iou > NMS_THRESH) | onehot, NEG, sc)
        return 0

    jax.lax.fori_loop(0, DET, step, 0)


@jax.jit
def kernel(boxes, scores):
    pad = ROWS * LANES - N
    x1 = jnp.pad(boxes[:, 0], (0, pad)).reshape(ROWS, LANES)
    y1 = jnp.pad(boxes[:, 1], (0, pad)).reshape(ROWS, LANES)
    x2 = jnp.pad(boxes[:, 2], (0, pad)).reshape(ROWS, LANES)
    y2 = jnp.pad(boxes[:, 3], (0, pad)).reshape(ROWS, LANES)
    s = jnp.pad(scores, (0, pad)).reshape(ROWS, LANES)

    out = pl.pallas_call(
        _nms_body,
        out_shape=jax.ShapeDtypeStruct((DET, LANES), jnp.float32),
        scratch_shapes=[pltpu.VMEM((ROWS, LANES), jnp.float32)],
    )(x1, y1, x2, y2, s)
    return out[:, :5]


# speedup vs baseline: 1.5320x; 1.2749x over previous
import jax
import jax.numpy as jnp
from jax.experimental import pallas as pl
from jax.experimental.pallas import tpu as pltpu

N = 20000
DET = 100
SCORE_THRESH = 0.05
NMS_THRESH = 0.5
NEG = -1e9

ROWS = 160
LANES = 128
BIGF = 3e7


def _nms_body(x1_ref, y1_ref, x2_ref, y2_ref, s_ref, out_ref, sc_ref):
    lane = jax.lax.broadcasted_iota(jnp.int32, (1, LANES), 1)
    lane_f = lane.astype(jnp.float32)
    row160_f = jax.lax.broadcasted_iota(jnp.int32, (ROWS, 1), 0).astype(
        jnp.float32)
    flat_iota_f = (
        jax.lax.broadcasted_iota(jnp.int32, (ROWS, LANES), 0) * LANES
        + jax.lax.broadcasted_iota(jnp.int32, (ROWS, LANES), 1)
    ).astype(jnp.float32)

    sc_ref[...] = jnp.where(s_ref[...] > SCORE_THRESH, s_ref[...], NEG)

    def step(i, _):
        sc = sc_ref[...]
        x1 = x1_ref[...]
        y1 = y1_ref[...]
        x2 = x2_ref[...]
        y2 = y2_ref[...]

        pls = jnp.max(sc, axis=0, keepdims=True)
        rowhit = jnp.min(jnp.where(sc == pls, row160_f, BIGF), axis=0,
                         keepdims=True)
        flat = rowhit * LANES + lane_f
        m = jnp.max(pls, axis=1, keepdims=True)
        idx = jnp.min(jnp.where(pls == m, flat, BIGF), axis=1,
                      keepdims=True)
        onehot = flat_iota_f == idx
        def pick(v):
            return jnp.max(jnp.max(jnp.where(onehot, v, -1e30), axis=0,
                                   keepdims=True), axis=1, keepdims=True)
        bx1 = pick(x1)
        by1 = pick(y1)
        bx2 = pick(x2)
        by2 = pick(y2)

        valid = (m > SCORE_THRESH).astype(jnp.float32)
        out_ref[pl.ds(i, 1), :] = jnp.where(
            lane == 0, bx1,
            jnp.where(lane == 1, by1,
            jnp.where(lane == 2, bx2,
            jnp.where(lane == 3, by2, m)))) * valid

        inter = (jnp.maximum(jnp.minimum(bx2, x2) - jnp.maximum(bx1, x1), 0.0)
                 * jnp.maximum(jnp.minimum(by2, y2) - jnp.maximum(by1, y1), 0.0))
        barea = (bx2 - bx1) * (by2 - by1)
        area = (x2 - x1) * (y2 - y1)
        iou = inter / (barea + area - inter + 1e-9)
        sc_ref[...] = jnp.where((iou > NMS_THRESH) | onehot, NEG, sc)
        return 0

    jax.lax.fori_loop(0, DET, step, 0)


@jax.jit
def kernel(boxes, scores):
    pad = ROWS * LANES - N
    x1 = jnp.pad(boxes[:, 0], (0, pad)).reshape(ROWS, LANES)
    y1 = jnp.pad(boxes[:, 1], (0, pad)).reshape(ROWS, LANES)
    x2 = jnp.pad(boxes[:, 2], (0, pad)).reshape(ROWS, LANES)
    y2 = jnp.pad(boxes[:, 3], (0, pad)).reshape(ROWS, LANES)
    s = jnp.pad(scores, (0, pad)).reshape(ROWS, LANES)

    out = pl.pallas_call(
        _nms_body,
        out_shape=jax.ShapeDtypeStruct((DET, LANES), jnp.float32),
        scratch_shapes=[pltpu.VMEM((ROWS, LANES), jnp.float32)],
    )(x1, y1, x2, y2, s)
    return out[:, :5]


# speedup vs baseline: 1.5368x; 1.0031x over previous
import jax
import jax.numpy as jnp
from jax.experimental import pallas as pl
from jax.experimental.pallas import tpu as pltpu

N = 20000
DET = 100
SCORE_THRESH = 0.05
NMS_THRESH = 0.5
NEG = -1e9

ROWS = 160
LANES = 128
BIGF = 3e7


def _nms_body(x1_ref, y1_ref, x2_ref, y2_ref, s_ref, out_ref, sc_ref):
    lane = jax.lax.broadcasted_iota(jnp.int32, (1, LANES), 1)
    lane_f = lane.astype(jnp.float32)
    row160_f = jax.lax.broadcasted_iota(jnp.int32, (ROWS, 1), 0).astype(
        jnp.float32)
    flat_iota_f = (
        jax.lax.broadcasted_iota(jnp.int32, (ROWS, LANES), 0) * LANES
        + jax.lax.broadcasted_iota(jnp.int32, (ROWS, LANES), 1)
    ).astype(jnp.float32)

    sc0 = jnp.where(s_ref[...] > SCORE_THRESH, s_ref[...], NEG)

    def _one(i, sc):
        x1 = x1_ref[...]
        y1 = y1_ref[...]
        x2 = x2_ref[...]
        y2 = y2_ref[...]

        pls = jnp.max(sc, axis=0, keepdims=True)
        rowhit = jnp.min(jnp.where(sc == pls, row160_f, BIGF), axis=0,
                         keepdims=True)
        flat = rowhit * LANES + lane_f
        m = jnp.max(pls, axis=1, keepdims=True)
        idx = jnp.min(jnp.where(pls == m, flat, BIGF), axis=1,
                      keepdims=True)
        onehot = flat_iota_f == idx
        def pick(v):
            return jnp.max(jnp.max(jnp.where(onehot, v, -1e30), axis=0,
                                   keepdims=True), axis=1, keepdims=True)
        bx1 = pick(x1)
        by1 = pick(y1)
        bx2 = pick(x2)
        by2 = pick(y2)

        valid = (m > SCORE_THRESH).astype(jnp.float32)
        out_ref[pl.ds(i, 1), :] = jnp.where(
            lane == 0, bx1,
            jnp.where(lane == 1, by1,
            jnp.where(lane == 2, bx2,
            jnp.where(lane == 3, by2, m)))) * valid

        inter = (jnp.maximum(jnp.minimum(bx2, x2) - jnp.maximum(bx1, x1), 0.0)
                 * jnp.maximum(jnp.minimum(by2, y2) - jnp.maximum(by1, y1), 0.0))
        barea = (bx2 - bx1) * (by2 - by1)
        area = (x2 - x1) * (y2 - y1)
        iou = inter / (barea + area - inter + 1e-9)
        return jnp.where((iou > NMS_THRESH) | onehot, NEG, sc)

    def dstep(k, sc):
        sc = _one(2 * k, sc)
        return _one(2 * k + 1, sc)

    jax.lax.fori_loop(0, DET // 2, dstep, sc0)


@jax.jit
def kernel(boxes, scores):
    pad = ROWS * LANES - N
    x1 = jnp.pad(boxes[:, 0], (0, pad)).reshape(ROWS, LANES)
    y1 = jnp.pad(boxes[:, 1], (0, pad)).reshape(ROWS, LANES)
    x2 = jnp.pad(boxes[:, 2], (0, pad)).reshape(ROWS, LANES)
    y2 = jnp.pad(boxes[:, 3], (0, pad)).reshape(ROWS, LANES)
    s = jnp.pad(scores, (0, pad)).reshape(ROWS, LANES)

    out = pl.pallas_call(
        _nms_body,
        out_shape=jax.ShapeDtypeStruct((DET, LANES), jnp.float32),
        scratch_shapes=[pltpu.VMEM((ROWS, LANES), jnp.float32)],
    )(x1, y1, x2, y2, s)
    return out[:, :5]


# speedup vs baseline: 1.6319x; 1.0619x over previous
import jax
import jax.numpy as jnp
from jax.experimental import pallas as pl
from jax.experimental.pallas import tpu as pltpu

N = 20000
DET = 100
SCORE_THRESH = 0.05
NMS_THRESH = 0.5
NEG = -1e9

ROWS = 160
LANES = 128
BIGF = 3e7


def _nms_body(x1_ref, y1_ref, x2_ref, y2_ref, s_ref, out_ref, sc_ref):
    lane = jax.lax.broadcasted_iota(jnp.int32, (1, LANES), 1)
    lane_f = lane.astype(jnp.float32)
    row160_f = jax.lax.broadcasted_iota(jnp.int32, (ROWS, 1), 0).astype(
        jnp.float32)
    flat_iota_f = (
        jax.lax.broadcasted_iota(jnp.int32, (ROWS, LANES), 0) * LANES
        + jax.lax.broadcasted_iota(jnp.int32, (ROWS, LANES), 1)
    ).astype(jnp.float32)

    sc0 = jnp.where(s_ref[...] > SCORE_THRESH, s_ref[...], NEG)

    def _one(i, sc):
        x1 = x1_ref[...]
        y1 = y1_ref[...]
        x2 = x2_ref[...]
        y2 = y2_ref[...]

        pls = jnp.max(sc, axis=0, keepdims=True)
        rowhit = jnp.min(jnp.where(sc == pls, row160_f, BIGF), axis=0,
                         keepdims=True)
        flat = rowhit * LANES + lane_f
        m = jnp.max(pls, axis=1, keepdims=True)
        idx = jnp.min(jnp.where(pls == m, flat, BIGF), axis=1,
                      keepdims=True)
        rowsel = row160_f == rowhit
        def plane(v):
            return jnp.max(jnp.where(rowsel, v, -1e30), axis=0, keepdims=True)
        pbx1 = plane(x1)
        pby1 = plane(y1)
        pbx2 = plane(x2)
        pby2 = plane(y2)
        lanewin = flat == idx
        def pick(v):
            return jnp.max(jnp.where(lanewin, v, -1e30), axis=1, keepdims=True)
        bx1 = pick(pbx1)
        by1 = pick(pby1)
        bx2 = pick(pbx2)
        by2 = pick(pby2)
        onehot = flat_iota_f == idx

        valid = (m > SCORE_THRESH).astype(jnp.float32)
        out_ref[pl.ds(i, 1), :] = jnp.where(
            lane == 0, bx1,
            jnp.where(lane == 1, by1,
            jnp.where(lane == 2, bx2,
            jnp.where(lane == 3, by2, m)))) * valid

        inter = (jnp.maximum(jnp.minimum(bx2, x2) - jnp.maximum(bx1, x1), 0.0)
                 * jnp.maximum(jnp.minimum(by2, y2) - jnp.maximum(by1, y1), 0.0))
        barea = (bx2 - bx1) * (by2 - by1)
        area = (x2 - x1) * (y2 - y1)
        iou = inter / (barea + area - inter + 1e-9)
        return jnp.where((iou > NMS_THRESH) | onehot, NEG, sc)

    def dstep(k, sc):
        sc = _one(2 * k, sc)
        return _one(2 * k + 1, sc)

    jax.lax.fori_loop(0, DET // 2, dstep, sc0)


@jax.jit
def kernel(boxes, scores):
    pad = ROWS * LANES - N
    x1 = jnp.pad(boxes[:, 0], (0, pad)).reshape(ROWS, LANES)
    y1 = jnp.pad(boxes[:, 1], (0, pad)).reshape(ROWS, LANES)
    x2 = jnp.pad(boxes[:, 2], (0, pad)).reshape(ROWS, LANES)
    y2 = jnp.pad(boxes[:, 3], (0, pad)).reshape(ROWS, LANES)
    s = jnp.pad(scores, (0, pad)).reshape(ROWS, LANES)

    out = pl.pallas_call(
        _nms_body,
        out_shape=jax.ShapeDtypeStruct((DET, LANES), jnp.float32),
        scratch_shapes=[pltpu.VMEM((ROWS, LANES), jnp.float32)],
    )(x1, y1, x2, y2, s)
    return out[:, :5]
